# Initial kernel scaffold; baseline (speedup 1.0000x reference)
import jax
import jax.numpy as jnp
from jax.experimental import pallas as pl


def kernel(feature_vectors, W1, b1, W2, b2, Wc, bc, Wr, br):
    raise NotImplementedError("write your pallas kernel here")


# speedup vs baseline: 1.0412x; 1.0412x over previous
import jax
import jax.numpy as jnp
from jax.experimental import pallas as pl

P = 7
IN_DIM = 256 * P * P
HID = 1024
HEADS = 16

BM = 128


def _dot(a, b):
    return jax.lax.dot_general(
        a, b, (((1,), (0,)), ((), ())), preferred_element_type=jnp.float32
    )


def _mlp_kernel(x_ref, w1_ref, b1_ref, w2_ref, b2_ref, wh_ref, bh_ref, out_ref):
    x = x_ref[...].astype(jnp.bfloat16)
    h = _dot(x, w1_ref[...]) + b1_ref[...]
    h = jnp.maximum(h, 0.0).astype(jnp.bfloat16)
    h = _dot(h, w2_ref[...]) + b2_ref[...]
    h = jnp.maximum(h, 0.0).astype(jnp.bfloat16)
    out_ref[...] = _dot(h, wh_ref[...]) + bh_ref[...]


def kernel(feature_vectors, W1, b1, W2, b2, Wc, bc, Wr, br):
    m = feature_vectors.shape[0]
    wh = jnp.concatenate([Wc, Wr], axis=1).astype(jnp.bfloat16)
    bh = jnp.concatenate([bc, br]).reshape(1, HEADS)
    w1 = W1.astype(jnp.bfloat16)
    w2 = W2.astype(jnp.bfloat16)

    out = pl.pallas_call(
        _mlp_kernel,
        grid=(pl.cdiv(m, BM),),
        in_specs=[
            pl.BlockSpec((BM, IN_DIM), lambda i: (i, 0)),
            pl.BlockSpec((IN_DIM, HID), lambda i: (0, 0)),
            pl.BlockSpec((1, HID), lambda i: (0, 0)),
            pl.BlockSpec((HID, HID), lambda i: (0, 0)),
            pl.BlockSpec((1, HID), lambda i: (0, 0)),
            pl.BlockSpec((HID, HEADS), lambda i: (0, 0)),
            pl.BlockSpec((1, HEADS), lambda i: (0, 0)),
        ],
        out_specs=pl.BlockSpec((BM, HEADS), lambda i: (i, 0)),
        out_shape=jax.ShapeDtypeStruct((m, HEADS), jnp.float32),
    )(
        feature_vectors,
        w1,
        b1.reshape(1, HID),
        w2,
        b2.reshape(1, HID),
        wh,
        bh,
    )
    return out[:, :4], out[:, 4:HEADS]


# speedup vs baseline: 1.1068x; 1.0630x over previous
import jax
import jax.numpy as jnp
from jax.experimental import pallas as pl

P = 7
IN_DIM = 256 * P * P
HID = 1024
HEADS = 16

BM = 256


def _dot(a, b):
    return jax.lax.dot_general(
        a, b, (((1,), (0,)), ((), ())), preferred_element_type=jnp.float32
    )


def _mlp_kernel(x_ref, w1_ref, b1_ref, w2_ref, b2_ref, wh_ref, bh_ref, out_ref):
    x = x_ref[...].astype(jnp.bfloat16)
    h = _dot(x, w1_ref[...]) + b1_ref[...]
    h = jnp.maximum(h, 0.0).astype(jnp.bfloat16)
    h = _dot(h, w2_ref[...]) + b2_ref[...]
    h = jnp.maximum(h, 0.0).astype(jnp.bfloat16)
    out_ref[...] = _dot(h, wh_ref[...]) + bh_ref[...]


def kernel(feature_vectors, W1, b1, W2, b2, Wc, bc, Wr, br):
    m = feature_vectors.shape[0]
    wh = jnp.concatenate([Wc, Wr], axis=1).astype(jnp.bfloat16)
    bh = jnp.concatenate([bc, br]).reshape(1, HEADS)
    w1 = W1.astype(jnp.bfloat16)
    w2 = W2.astype(jnp.bfloat16)

    out = pl.pallas_call(
        _mlp_kernel,
        grid=(pl.cdiv(m, BM),),
        in_specs=[
            pl.BlockSpec((BM, IN_DIM), lambda i: (i, 0)),
            pl.BlockSpec((IN_DIM, HID), lambda i: (0, 0)),
            pl.BlockSpec((1, HID), lambda i: (0, 0)),
            pl.BlockSpec((HID, HID), lambda i: (0, 0)),
            pl.BlockSpec((1, HID), lambda i: (0, 0)),
            pl.BlockSpec((HID, HEADS), lambda i: (0, 0)),
            pl.BlockSpec((1, HEADS), lambda i: (0, 0)),
        ],
        out_specs=pl.BlockSpec((BM, HEADS), lambda i: (i, 0)),
        out_shape=jax.ShapeDtypeStruct((m, HEADS), jnp.float32),
    )(
        feature_vectors,
        w1,
        b1.reshape(1, HID),
        w2,
        b2.reshape(1, HID),
        wh,
        bh,
    )
    return out[:, :4], out[:, 4:HEADS]
